# baseline (speedup 1.0000x reference)
import functools

import jax
import jax.numpy as jnp
from jax import lax
from jax.experimental import pallas as pl
from jax.experimental.pallas import tpu as pltpu
import jax.experimental.pallas.tpu_sc as plsc

_CONV, _OUT, _NB = 32, 10, 32
_BLK = 1024
_C = 128
_NSUB = 16
_NCORE = 2


def _rup(x, m):
    return (x + m - 1) // m * m


def _lrelu(t):
    return jnp.where(t > 0, t, 0.01 * t)


_SUP = 8
_SUPE = _SUP * _C
_HALF = _SUP // 2 * _C


@functools.lru_cache(maxsize=None)
def _make_spmm(n_pad, e_pad, esplit):
    eps = e_pad // (_NSUB * _NCORE) if esplit else e_pad // _NSUB
    nblocks = eps // _SUPE
    nz = n_pad // (_NSUB * _C)
    rows_per_sub = n_pad // _NSUB
    erow_total = e_pad // _C
    mesh = plsc.VectorSubcoreMesh(
        core_axis_name="c", subcore_axis_name="s",
        num_cores=_NCORE, num_subcores=_NSUB)

    @functools.partial(
        pl.kernel,
        out_type=jax.ShapeDtypeStruct((2 * n_pad, 16), jnp.float32),
        mesh=mesh,
        scratch_types=[
            pltpu.VMEM_SHARED((n_pad, 16), jnp.float32),
            pltpu.VMEM((_SUPE,), jnp.int32),
            pltpu.VMEM((_SUP, _C), jnp.int32),
            pltpu.VMEM((_SUPE,), jnp.float32),
            pltpu.VMEM((_C, 16), jnp.float32),
            pltpu.VMEM((_C, 16), jnp.float32),
            pltpu.VMEM((_C, 16), jnp.float32),
            pltpu.SemaphoreType.DMA,
            pltpu.SemaphoreType.DMA,
            pltpu.SemaphoreType.DMA,
        ],
        compiler_params=pltpu.CompilerParams(use_tc_tiling_on_sc=False),
    )
    def spmm(u0, u1, rows2, cols, vals, y, acc, colv, rowv8, valv, gbufa,
             gbufb, zbuf, semi, sema, semb):
        cid = lax.axis_index("c")
        sid = lax.axis_index("s")
        rbase = sid * rows_per_sub
        for r in range(_C):
            zbuf[r] = jnp.zeros((16,), jnp.float32)
        for j in range(nz):
            pltpu.sync_copy(zbuf, acc.at[pl.ds(rbase + j * _C, _C)])
        plsc.subcore_barrier()

        ebase = sid * eps + cid * (e_pad // 2 if esplit else 0)
        ebrow = sid * (eps // _C) + cid * (e_pad // 2 // _C if esplit else 0)
        gbufs = (gbufa, gbufb)
        gsems = (sema, semb)

        def stage(eoff, eroff, half):
            co = half * _HALF
            return [
                pltpu.async_copy(cols.at[pl.ds(eoff, _HALF)],
                                 colv.at[pl.ds(co, _HALF)], semi),
                pltpu.async_copy(vals.at[pl.ds(eoff, _HALF)],
                                 valv.at[pl.ds(co, _HALF)], semi),
                pltpu.async_copy(rows2.at[pl.ds(eroff, _SUP // 2)],
                                 rowv8.at[pl.ds(half * (_SUP // 2), _SUP // 2)],
                                 semi),
            ]

        def issue_gather(k, buf, sem):
            idxref = colv.at[pl.ds(k * _C, _C)]

            @pl.when(cid == 0)
            def _():
                pltpu.async_copy(u0.at[idxref], buf, sem)

            @pl.when(cid == 1)
            def _():
                pltpu.async_copy(u1.at[idxref], buf, sem)

            return pltpu.make_async_copy(u0.at[idxref], buf, sem)

        def process(k, buf):
            base = k * _C
            for g in range(_C // 16):
                vv = valv[pl.ds(base + g * 16, 16)]
                for l in range(16):
                    e = g * 16 + l
                    buf[e] = buf[e] * vv[l]
            pltpu.sync_copy(buf, acc.at[rowv8.at[k]], add=True)

        for dd in stage(ebase, ebrow, 0):
            dd.wait()
        issue_gather(0, gbufa, sema)

        def block(b, carry):
            eoff = ebase + b * _SUPE
            eroff = ebrow + b * _SUP
            hb_descs = stage(eoff + _HALF, eroff + _SUP // 2, 1)
            nxt = jnp.minimum(eoff + _SUPE, e_pad - _HALF)
            nxtr = jnp.minimum(eroff + _SUP, erow_total - _SUP // 2)
            na_descs = None
            d = [pltpu.make_async_copy(u0.at[colv.at[pl.ds(0, _C)]],
                                       gbufa, sema)] + [None] * (_SUP - 1)
            for k in range(_SUP):
                buf = gbufs[k % 2]
                if k == _SUP // 2 - 1:
                    for dd in hb_descs:
                        dd.wait()
                if k < _SUP - 1:
                    d[k + 1] = issue_gather(k + 1, gbufs[(k + 1) % 2],
                                            gsems[(k + 1) % 2])
                if k == _SUP // 2:
                    na_descs = stage(nxt, nxtr, 0)
                if k == _SUP - 1:
                    for dd in na_descs:
                        dd.wait()
                    issue_gather(0, gbufa, sema)
                d[k].wait()
                process(k, buf)
            return carry

        lax.fori_loop(0, nblocks, block, 0)
        pltpu.make_async_copy(u0.at[colv.at[pl.ds(0, _C)]], gbufa, sema).wait()

        plsc.subcore_barrier()
        ybase = cid * n_pad + rbase
        for j in range(nz):
            pltpu.sync_copy(acc.at[pl.ds(rbase + j * _C, _C)],
                            y.at[pl.ds(ybase + j * _C, _C)])

    return spmm


@functools.lru_cache(maxsize=None)
def _make_edgepool(n_pad, e_pad):
    sup = 4
    supe = sup * _C
    half = supe // 2
    epw = e_pad // (_NSUB * _NCORE)
    nblocks = epw // supe
    mesh = plsc.VectorSubcoreMesh(
        core_axis_name="c", subcore_axis_name="s",
        num_cores=_NCORE, num_subcores=_NSUB)

    @functools.partial(
        pl.kernel,
        out_type=jax.ShapeDtypeStruct((2 * _NSUB * _NB * 16,), jnp.float32),
        mesh=mesh,
        scratch_types=[
            pltpu.VMEM((supe,), jnp.int32),
            pltpu.VMEM((supe,), jnp.int32),
            pltpu.VMEM((supe,), jnp.float32),
            pltpu.VMEM((_C, 16), jnp.float32),
            pltpu.VMEM((_C, 16), jnp.float32),
            pltpu.VMEM((_C, 16), jnp.int32),
            pltpu.VMEM((_C, 16), jnp.int32),
            pltpu.VMEM((_NB * 16,), jnp.float32),
            pltpu.SemaphoreType.DMA,
            pltpu.SemaphoreType.DMA,
            pltpu.SemaphoreType.DMA,
        ],
        compiler_params=pltpu.CompilerParams(use_tc_tiling_on_sc=False),
    )
    def epool(u, bt, rows, cols, vals, y, colv, rowv, valv, ga, gb, ba,
              bb, acct, semi, sema, semb):
        cid = lax.axis_index("c")
        sid = lax.axis_index("s")
        iota = lax.broadcasted_iota(jnp.int32, (16,), 0)
        for r in range(_NB):
            acct[pl.ds(r * 16, 16)] = jnp.zeros((16,), jnp.float32)

        ebase = (cid * _NSUB + sid) * epw
        gbufs = (ga, gb)
        bbufs = (ba, bb)
        gsems = (sema, semb)

        def stage(eoff, h):
            co = h * half
            return [
                pltpu.async_copy(cols.at[pl.ds(eoff, half)],
                                 colv.at[pl.ds(co, half)], semi),
                pltpu.async_copy(rows.at[pl.ds(eoff, half)],
                                 rowv.at[pl.ds(co, half)], semi),
                pltpu.async_copy(vals.at[pl.ds(eoff, half)],
                                 valv.at[pl.ds(co, half)], semi),
            ]

        def issue_gather(k):
            p = k % 2
            pltpu.async_copy(u.at[colv.at[pl.ds(k * _C, _C)]], gbufs[p],
                             gsems[p])
            pltpu.async_copy(bt.at[rowv.at[pl.ds(k * _C, _C)]], bbufs[p],
                             gsems[p])
            return [pltpu.make_async_copy(u.at[colv.at[pl.ds(k * _C, _C)]],
                                          gbufs[p], gsems[p]),
                    pltpu.make_async_copy(bt.at[rowv.at[pl.ds(k * _C, _C)]],
                                          bbufs[p], gsems[p])]

        def process(k):
            p = k % 2
            gbuf, bbuf = gbufs[p], bbufs[p]
            base = k * _C
            for g in range(_C // 16):
                vv = valv[pl.ds(base + g * 16, 16)]
                for l in range(16):
                    e = g * 16 + l
                    b0 = bbuf[e][0]
                    plsc.addupdate(acct.at[pl.ds(b0 * 16, 16)],
                                   gbuf[e] * vv[l])

        for dd in stage(ebase, 0):
            dd.wait()
        issue_gather(0)

        def block(b, carry):
            eoff = ebase + b * supe
            hb_descs = stage(eoff + half, 1)
            nxt = jnp.minimum(eoff + supe, e_pad - half)
            na_descs = None
            ds = {0: [pltpu.make_async_copy(u.at[colv.at[pl.ds(0, _C)]],
                                            ga, sema),
                      pltpu.make_async_copy(bt.at[rowv.at[pl.ds(0, _C)]],
                                            ba, sema)]}
            for k in range(sup):
                if k == sup // 2 - 1:
                    for dd in hb_descs:
                        dd.wait()
                if k < sup - 1:
                    ds[k + 1] = issue_gather(k + 1)
                if k == sup // 2:
                    na_descs = stage(nxt, 0)
                if k == sup - 1:
                    for dd in na_descs:
                        dd.wait()
                    issue_gather(0)
                for dd in ds[k]:
                    dd.wait()
                process(k)
            return carry

        lax.fori_loop(0, nblocks, block, 0)
        pltpu.make_async_copy(u.at[colv.at[pl.ds(0, _C)]], ga, sema).wait()
        pltpu.make_async_copy(bt.at[rowv.at[pl.ds(0, _C)]], ba, sema).wait()

        wid = cid * _NSUB + sid
        pltpu.sync_copy(acct, y.at[pl.ds(wid * _NB * 16, _NB * 16)])

    return epool


def _full(shape):
    return pl.BlockSpec(shape, lambda i: (0, 0))


def _rowblk(w):
    return pl.BlockSpec((_BLK, w), lambda i: (i, 0))


def _rowblk_hi(w, npb):
    return pl.BlockSpec((_BLK, w), lambda i: (i + npb, 0))


def _stage1_body(x, wa, wblo, wbhi, b, a_out, ulo, uhi):
    xb = x[...]
    a_out[...] = jnp.dot(xb, wa[...], preferred_element_type=jnp.float32) + b[...]
    ulo[...] = jnp.dot(xb, wblo[...], preferred_element_type=jnp.float32)
    uhi[...] = jnp.dot(xb, wbhi[...], preferred_element_type=jnp.float32)


@functools.lru_cache(maxsize=None)
def _make_stage1(n_pad):
    return pl.pallas_call(
        _stage1_body,
        grid=(n_pad // _BLK,),
        in_specs=[_rowblk(64), _full((64, 32)), _full((64, 16)),
                  _full((64, 16)), _full((1, 32))],
        out_specs=[_rowblk(32), _rowblk(16), _rowblk(16)],
        out_shape=[jax.ShapeDtypeStruct((n_pad, 32), jnp.float32),
                   jax.ShapeDtypeStruct((n_pad, 16), jnp.float32),
                   jax.ShapeDtypeStruct((n_pad, 16), jnp.float32)],
    )


def _stage2_body(a_in, ylo, yhi, wa, wblo, wbhi, b, a_out, ulo, uhi):
    h = _lrelu(a_in[...] + jnp.concatenate([ylo[...], yhi[...]], axis=1))
    a_out[...] = jnp.dot(h, wa[...], preferred_element_type=jnp.float32) + b[...]
    ulo[...] = jnp.dot(h, wblo[...], preferred_element_type=jnp.float32)
    uhi[...] = jnp.dot(h, wbhi[...], preferred_element_type=jnp.float32)


@functools.lru_cache(maxsize=None)
def _make_stage2(n_pad):
    npb = n_pad // _BLK
    return pl.pallas_call(
        _stage2_body,
        grid=(npb,),
        in_specs=[_rowblk(32), _rowblk(16), _rowblk_hi(16, npb),
                  _full((32, 32)), _full((32, 16)), _full((32, 16)),
                  _full((1, 32))],
        out_specs=[_rowblk(32), _rowblk(16), _rowblk(16)],
        out_shape=[jax.ShapeDtypeStruct((n_pad, 32), jnp.float32),
                   jax.ShapeDtypeStruct((n_pad, 16), jnp.float32),
                   jax.ShapeDtypeStruct((n_pad, 16), jnp.float32)],
    )


def _stage3_body(a_in, ylo, yhi, wa, wb, b, a_out, u_out):
    h = _lrelu(a_in[...] + jnp.concatenate([ylo[...], yhi[...]], axis=1))
    a_out[...] = jnp.dot(h, wa[...], preferred_element_type=jnp.float32) + b[...]
    u_out[...] = jnp.dot(h, wb[...], preferred_element_type=jnp.float32)


@functools.lru_cache(maxsize=None)
def _make_stage3(n_pad):
    npb = n_pad // _BLK
    return pl.pallas_call(
        _stage3_body,
        grid=(npb,),
        in_specs=[_rowblk(32), _rowblk(16), _rowblk_hi(16, npb),
                  _full((32, 16)), _full((32, 16)), _full((1, 16))],
        out_specs=[_rowblk(16), _rowblk(16)],
        out_shape=[jax.ShapeDtypeStruct((n_pad, 16), jnp.float32),
                   jax.ShapeDtypeStruct((n_pad, 16), jnp.float32)],
    )


def _gmp_body(a3, bid, sum_out, cnt_out, sacc, cacc):
    i = pl.program_id(0)
    n = pl.num_programs(0)
    oh = (bid[...] == lax.broadcasted_iota(jnp.int32, (_BLK, _NB), 1)
          ).astype(jnp.float32)
    dn = (((0,), (0,)), ((), ()))
    s = lax.dot_general(oh, a3[...], dn, preferred_element_type=jnp.float32)
    c = lax.dot_general(oh, jnp.ones((_BLK, 16), jnp.float32), dn,
                        preferred_element_type=jnp.float32)

    @pl.when(i == 0)
    def _():
        sacc[...] = s
        cacc[...] = c

    @pl.when(i > 0)
    def _():
        sacc[...] += s
        cacc[...] += c

    @pl.when(i == n - 1)
    def _():
        sum_out[...] = sacc[...]
        cnt_out[...] = cacc[...]


@functools.lru_cache(maxsize=None)
def _make_gmp(n_pad):
    return pl.pallas_call(
        _gmp_body,
        grid=(n_pad // _BLK,),
        in_specs=[_rowblk(16), pl.BlockSpec((_BLK, 1), lambda i: (i, 0))],
        out_specs=[pl.BlockSpec((_NB, 16), lambda i: (0, 0)),
                   pl.BlockSpec((_NB, 16), lambda i: (0, 0))],
        out_shape=[jax.ShapeDtypeStruct((_NB, 16), jnp.float32),
                   jax.ShapeDtypeStruct((_NB, 16), jnp.float32)],
        scratch_shapes=[pltpu.VMEM((_NB, 16), jnp.float32),
                        pltpu.VMEM((_NB, 16), jnp.float32)],
    )


def _epsum(e):
    acc = e[pl.ds(0, _NB), :]
    for t in range(1, 2 * _NSUB):
        acc = acc + e[pl.ds(t * _NB, _NB), :]
    return acc


def _final_body(s0, c0, e0, s1, c1, e1, s2, c2, e2, w0, w1, w2, b, out):
    m0 = (s0[...] + _epsum(e0)) / jnp.maximum(c0[...], 1.0)
    m1 = (s1[...] + _epsum(e1)) / jnp.maximum(c1[...], 1.0)
    m2 = (s2[...] + _epsum(e2)) / jnp.maximum(c2[...], 1.0)
    lg = (jnp.dot(m0, w0[...], preferred_element_type=jnp.float32)
          + jnp.dot(m1, w1[...], preferred_element_type=jnp.float32)
          + jnp.dot(m2, w2[...], preferred_element_type=jnp.float32)
          + b[...])
    col = lax.broadcasted_iota(jnp.int32, (_NB, 16), 1)
    lg = jnp.where(col < _OUT, lg, -1e30)
    mx = jnp.max(lg, axis=1, keepdims=True)
    e = jnp.exp(lg - mx)
    out[...] = e / jnp.sum(e, axis=1, keepdims=True)


@functools.lru_cache(maxsize=None)
def _make_final():
    return pl.pallas_call(
        _final_body,
        out_shape=jax.ShapeDtypeStruct((_NB, 16), jnp.float32),
    )


def _branch(x, idx, val, bid, w1, c1, w2, c2, w3, c3):
    n, d = x.shape
    nnz = val.shape[0]
    n_pad = _rup(n, _NSUB * _C)
    e_pad = _rup(nnz, _NSUB * _NCORE * _SUPE)

    xp = jnp.pad(x, ((0, n_pad - n), (0, 0)))
    rows = jnp.pad(idx[0], (0, e_pad - nnz)).astype(jnp.int32)
    rows2 = rows.reshape(-1, _C)
    cols = jnp.pad(idx[1], (0, e_pad - nnz)).astype(jnp.int32)
    vals = jnp.pad(val, (0, e_pad - nnz))
    bidp = jnp.pad(bid, (0, n_pad - n), constant_values=_NB)
    bidp = bidp.astype(jnp.int32).reshape(n_pad, 1)

    a1, u1lo, u1hi = _make_stage1(n_pad)(
        xp, w1[:d], w1[d:, :16], w1[d:, 16:], c1.reshape(1, 32))
    y1 = _make_spmm(n_pad, e_pad, False)(u1lo, u1hi, rows2, cols, vals)

    a2, u2lo, u2hi = _make_stage2(n_pad)(
        a1, y1, y1, w2[:_CONV], w2[_CONV:, :16], w2[_CONV:, 16:],
        c2.reshape(1, 32))
    y2 = _make_spmm(n_pad, e_pad, False)(u2lo, u2hi, rows2, cols, vals)

    w3a = jnp.pad(w3[:_CONV], ((0, 0), (0, 16 - _OUT)))
    w3b = jnp.pad(w3[_CONV:], ((0, 0), (0, 16 - _OUT)))
    b3p = jnp.pad(c3, (0, 16 - _OUT)).reshape(1, 16)
    a3, u3 = _make_stage3(n_pad)(a2, y2, y2, w3a, w3b, b3p)
    bt16 = jnp.tile(bidp, (1, 16))
    ep = _make_edgepool(n_pad, e_pad)(u3, bt16, rows, cols, vals)
    ep = ep.reshape(2 * _NSUB * _NB, 16)

    nsum, cnt = _make_gmp(n_pad)(a3, bidp)
    return nsum, cnt, ep


def kernel(X0, L0_indices, L0_values, batch0,
           X1, L1_indices, L1_values, batch1,
           X2, L2_indices, L2_values, batch2,
           W0_1, b0_1, W0_2, b0_2, W0_3, b0_3,
           W1_1, b1_1, W1_2, b1_2, W1_3, b1_3,
           W2_1, b2_1, W2_2, b2_2, W2_3, b2_3,
           Wf, bf):
    s0, c0, e0 = _branch(X0, L0_indices, L0_values, batch0,
                         W0_1, b0_1, W0_2, b0_2, W0_3, b0_3)
    s1, c1, e1 = _branch(X1, L1_indices, L1_values, batch1,
                         W1_1, b1_1, W1_2, b1_2, W1_3, b1_3)
    s2, c2, e2 = _branch(X2, L2_indices, L2_values, batch2,
                         W2_1, b2_1, W2_2, b2_2, W2_3, b2_3)

    wfp = [jnp.pad(Wf[10 * k:10 * (k + 1)], ((0, 6), (0, 6)))
           for k in range(3)]
    bfp = jnp.pad(bf, (0, 6)).reshape(1, 16)
    out = _make_final()(s0, c0, e0, s1, c1, e1, s2, c2, e2,
                        wfp[0], wfp[1], wfp[2], bfp)
    return out[:, :_OUT]


# speedup vs baseline: 1.0013x; 1.0013x over previous
import functools

import jax
import jax.numpy as jnp
from jax import lax
from jax.experimental import pallas as pl
from jax.experimental.pallas import tpu as pltpu
import jax.experimental.pallas.tpu_sc as plsc

_CONV, _OUT, _NB = 32, 10, 32
_BLK = 1024
_C = 128
_NSUB = 16
_NCORE = 2


def _rup(x, m):
    return (x + m - 1) // m * m


def _lrelu(t):
    return jnp.where(t > 0, t, 0.01 * t)


_SUP = 8
_SUPE = _SUP * _C
_HALF = _SUP // 2 * _C


@functools.lru_cache(maxsize=None)
def _make_spmm(n_pad, e_pad, esplit):
    eps = e_pad // (_NSUB * _NCORE) if esplit else e_pad // _NSUB
    nblocks = eps // _SUPE
    nz = n_pad // (_NSUB * _C)
    rows_per_sub = n_pad // _NSUB
    erow_total = e_pad // _C
    mesh = plsc.VectorSubcoreMesh(
        core_axis_name="c", subcore_axis_name="s",
        num_cores=_NCORE, num_subcores=_NSUB)

    @functools.partial(
        pl.kernel,
        out_type=jax.ShapeDtypeStruct((2 * n_pad, 16), jnp.float32),
        mesh=mesh,
        scratch_types=[
            pltpu.VMEM_SHARED((n_pad, 16), jnp.float32),
            pltpu.VMEM((_SUPE,), jnp.int32),
            pltpu.VMEM((_SUP, _C), jnp.int32),
            pltpu.VMEM((_SUPE,), jnp.float32),
            pltpu.VMEM((_C, 16), jnp.float32),
            pltpu.VMEM((_C, 16), jnp.float32),
            pltpu.VMEM((_C, 16), jnp.float32),
            pltpu.SemaphoreType.DMA,
            pltpu.SemaphoreType.DMA,
            pltpu.SemaphoreType.DMA,
        ],
        compiler_params=pltpu.CompilerParams(use_tc_tiling_on_sc=False),
    )
    def spmm(u0, u1, rows2, cols, vals, y, acc, colv, rowv8, valv, gbufa,
             gbufb, zbuf, semi, sema, semb):
        cid = lax.axis_index("c")
        sid = lax.axis_index("s")
        rbase = sid * rows_per_sub
        for r in range(_C):
            zbuf[r] = jnp.zeros((16,), jnp.float32)
        for j in range(nz):
            pltpu.sync_copy(zbuf, acc.at[pl.ds(rbase + j * _C, _C)])
        plsc.subcore_barrier()

        ebase = sid * eps + cid * (e_pad // 2 if esplit else 0)
        ebrow = sid * (eps // _C) + cid * (e_pad // 2 // _C if esplit else 0)
        gbufs = (gbufa, gbufb)
        gsems = (sema, semb)

        def stage(eoff, eroff, half):
            co = half * _HALF
            return [
                pltpu.async_copy(cols.at[pl.ds(eoff, _HALF)],
                                 colv.at[pl.ds(co, _HALF)], semi),
                pltpu.async_copy(vals.at[pl.ds(eoff, _HALF)],
                                 valv.at[pl.ds(co, _HALF)], semi),
                pltpu.async_copy(rows2.at[pl.ds(eroff, _SUP // 2)],
                                 rowv8.at[pl.ds(half * (_SUP // 2), _SUP // 2)],
                                 semi),
            ]

        def issue_gather(k, buf, sem):
            idxref = colv.at[pl.ds(k * _C, _C)]

            @pl.when(cid == 0)
            def _():
                pltpu.async_copy(u0.at[idxref], buf, sem)

            @pl.when(cid == 1)
            def _():
                pltpu.async_copy(u1.at[idxref], buf, sem)

            return pltpu.make_async_copy(u0.at[idxref], buf, sem)

        def process(k, buf):
            base = k * _C
            for g in range(_C // 16):
                vv = valv[pl.ds(base + g * 16, 16)]
                for l in range(16):
                    e = g * 16 + l
                    buf[e] = buf[e] * vv[l]
            pltpu.sync_copy(buf, acc.at[rowv8.at[k]], add=True)

        for dd in stage(ebase, ebrow, 0):
            dd.wait()
        issue_gather(0, gbufa, sema)

        def block(b, carry):
            eoff = ebase + b * _SUPE
            eroff = ebrow + b * _SUP
            hb_descs = stage(eoff + _HALF, eroff + _SUP // 2, 1)
            nxt = jnp.minimum(eoff + _SUPE, e_pad - _HALF)
            nxtr = jnp.minimum(eroff + _SUP, erow_total - _SUP // 2)
            na_descs = None
            d = [pltpu.make_async_copy(u0.at[colv.at[pl.ds(0, _C)]],
                                       gbufa, sema)] + [None] * (_SUP - 1)
            for k in range(_SUP):
                buf = gbufs[k % 2]
                if k == _SUP // 2 - 1:
                    for dd in hb_descs:
                        dd.wait()
                if k < _SUP - 1:
                    d[k + 1] = issue_gather(k + 1, gbufs[(k + 1) % 2],
                                            gsems[(k + 1) % 2])
                if k == _SUP // 2:
                    na_descs = stage(nxt, nxtr, 0)
                if k == _SUP - 1:
                    for dd in na_descs:
                        dd.wait()
                    issue_gather(0, gbufa, sema)
                d[k].wait()
                process(k, buf)
            return carry

        lax.fori_loop(0, nblocks, block, 0)
        pltpu.make_async_copy(u0.at[colv.at[pl.ds(0, _C)]], gbufa, sema).wait()

        plsc.subcore_barrier()
        ybase = cid * n_pad + rbase
        for j in range(nz):
            pltpu.sync_copy(acc.at[pl.ds(rbase + j * _C, _C)],
                            y.at[pl.ds(ybase + j * _C, _C)])

    return spmm


@functools.lru_cache(maxsize=None)
def _make_edgepool(n_pad, e_pad):
    sup = 4
    supe = sup * _C
    half = supe // 2
    epw = e_pad // (_NSUB * _NCORE)
    nblocks = epw // supe
    mesh = plsc.VectorSubcoreMesh(
        core_axis_name="c", subcore_axis_name="s",
        num_cores=_NCORE, num_subcores=_NSUB)

    @functools.partial(
        pl.kernel,
        out_type=jax.ShapeDtypeStruct((2 * _NSUB * _NB * 16,), jnp.float32),
        mesh=mesh,
        scratch_types=[
            pltpu.VMEM((supe,), jnp.int32),
            pltpu.VMEM((supe,), jnp.int32),
            pltpu.VMEM((supe,), jnp.float32),
            pltpu.VMEM((_C, 16), jnp.float32),
            pltpu.VMEM((_C, 16), jnp.float32),
            pltpu.VMEM((_C, 16), jnp.int32),
            pltpu.VMEM((_C, 16), jnp.int32),
            pltpu.VMEM((_NB * 16,), jnp.float32),
            pltpu.VMEM((_NB * 16,), jnp.float32),
            pltpu.VMEM((_NB * 16,), jnp.float32),
            pltpu.VMEM((_NB * 16,), jnp.float32),
            pltpu.SemaphoreType.DMA,
            pltpu.SemaphoreType.DMA,
            pltpu.SemaphoreType.DMA,
        ],
        compiler_params=pltpu.CompilerParams(use_tc_tiling_on_sc=False),
    )
    def epool(u, bt, rows, cols, vals, y, colv, rowv, valv, ga, gb, ba,
              bb, acct0, acct1, acct2, acct3, semi, sema, semb):
        cid = lax.axis_index("c")
        sid = lax.axis_index("s")
        accts = (acct0, acct1, acct2, acct3)
        for r in range(_NB):
            z = jnp.zeros((16,), jnp.float32)
            for a in accts:
                a[pl.ds(r * 16, 16)] = z

        ebase = (cid * _NSUB + sid) * epw
        gbufs = (ga, gb)
        bbufs = (ba, bb)
        gsems = (sema, semb)

        def stage(eoff, h):
            co = h * half
            return [
                pltpu.async_copy(cols.at[pl.ds(eoff, half)],
                                 colv.at[pl.ds(co, half)], semi),
                pltpu.async_copy(rows.at[pl.ds(eoff, half)],
                                 rowv.at[pl.ds(co, half)], semi),
                pltpu.async_copy(vals.at[pl.ds(eoff, half)],
                                 valv.at[pl.ds(co, half)], semi),
            ]

        def issue_gather(k):
            p = k % 2
            pltpu.async_copy(u.at[colv.at[pl.ds(k * _C, _C)]], gbufs[p],
                             gsems[p])
            pltpu.async_copy(bt.at[rowv.at[pl.ds(k * _C, _C)]], bbufs[p],
                             gsems[p])
            return [pltpu.make_async_copy(u.at[colv.at[pl.ds(k * _C, _C)]],
                                          gbufs[p], gsems[p]),
                    pltpu.make_async_copy(bt.at[rowv.at[pl.ds(k * _C, _C)]],
                                          bbufs[p], gsems[p])]

        def process(k):
            p = k % 2
            gbuf, bbuf = gbufs[p], bbufs[p]
            base = k * _C
            for g in range(_C // 16):
                vv = valv[pl.ds(base + g * 16, 16)]
                for l in range(16):
                    e = g * 16 + l
                    b0 = bbuf[e][0]
                    plsc.addupdate(accts[e % 4].at[pl.ds(b0 * 16, 16)],
                                   gbuf[e] * vv[l])

        for dd in stage(ebase, 0):
            dd.wait()
        issue_gather(0)

        def block(b, carry):
            eoff = ebase + b * supe
            hb_descs = stage(eoff + half, 1)
            nxt = jnp.minimum(eoff + supe, e_pad - half)
            na_descs = None
            ds = {0: [pltpu.make_async_copy(u.at[colv.at[pl.ds(0, _C)]],
                                            ga, sema),
                      pltpu.make_async_copy(bt.at[rowv.at[pl.ds(0, _C)]],
                                            ba, sema)]}
            for k in range(sup):
                if k == sup // 2 - 1:
                    for dd in hb_descs:
                        dd.wait()
                if k < sup - 1:
                    ds[k + 1] = issue_gather(k + 1)
                if k == sup // 2:
                    na_descs = stage(nxt, 0)
                if k == sup - 1:
                    for dd in na_descs:
                        dd.wait()
                    issue_gather(0)
                for dd in ds[k]:
                    dd.wait()
                process(k)
            return carry

        lax.fori_loop(0, nblocks, block, 0)
        pltpu.make_async_copy(u.at[colv.at[pl.ds(0, _C)]], ga, sema).wait()
        pltpu.make_async_copy(bt.at[rowv.at[pl.ds(0, _C)]], ba, sema).wait()

        for r in range(_NB):
            sl = pl.ds(r * 16, 16)
            acct0[sl] = (acct0[sl] + acct1[sl]) + (acct2[sl] + acct3[sl])
        wid = cid * _NSUB + sid
        pltpu.sync_copy(acct0, y.at[pl.ds(wid * _NB * 16, _NB * 16)])

    return epool


def _full(shape):
    return pl.BlockSpec(shape, lambda i: (0, 0))


def _rowblk(w):
    return pl.BlockSpec((_BLK, w), lambda i: (i, 0))


def _rowblk_hi(w, npb):
    return pl.BlockSpec((_BLK, w), lambda i: (i + npb, 0))


def _stage1_body(x, wa, wblo, wbhi, b, a_out, ulo, uhi):
    xb = x[...]
    a_out[...] = jnp.dot(xb, wa[...], preferred_element_type=jnp.float32) + b[...]
    ulo[...] = jnp.dot(xb, wblo[...], preferred_element_type=jnp.float32)
    uhi[...] = jnp.dot(xb, wbhi[...], preferred_element_type=jnp.float32)


@functools.lru_cache(maxsize=None)
def _make_stage1(n_pad):
    return pl.pallas_call(
        _stage1_body,
        grid=(n_pad // _BLK,),
        in_specs=[_rowblk(64), _full((64, 32)), _full((64, 16)),
                  _full((64, 16)), _full((1, 32))],
        out_specs=[_rowblk(32), _rowblk(16), _rowblk(16)],
        out_shape=[jax.ShapeDtypeStruct((n_pad, 32), jnp.float32),
                   jax.ShapeDtypeStruct((n_pad, 16), jnp.float32),
                   jax.ShapeDtypeStruct((n_pad, 16), jnp.float32)],
    )


def _stage2_body(a_in, ylo, yhi, wa, wblo, wbhi, b, a_out, ulo, uhi):
    h = _lrelu(a_in[...] + jnp.concatenate([ylo[...], yhi[...]], axis=1))
    a_out[...] = jnp.dot(h, wa[...], preferred_element_type=jnp.float32) + b[...]
    ulo[...] = jnp.dot(h, wblo[...], preferred_element_type=jnp.float32)
    uhi[...] = jnp.dot(h, wbhi[...], preferred_element_type=jnp.float32)


@functools.lru_cache(maxsize=None)
def _make_stage2(n_pad):
    npb = n_pad // _BLK
    return pl.pallas_call(
        _stage2_body,
        grid=(npb,),
        in_specs=[_rowblk(32), _rowblk(16), _rowblk_hi(16, npb),
                  _full((32, 32)), _full((32, 16)), _full((32, 16)),
                  _full((1, 32))],
        out_specs=[_rowblk(32), _rowblk(16), _rowblk(16)],
        out_shape=[jax.ShapeDtypeStruct((n_pad, 32), jnp.float32),
                   jax.ShapeDtypeStruct((n_pad, 16), jnp.float32),
                   jax.ShapeDtypeStruct((n_pad, 16), jnp.float32)],
    )


def _stage3_body(a_in, ylo, yhi, wa, wb, b, a_out, u_out):
    h = _lrelu(a_in[...] + jnp.concatenate([ylo[...], yhi[...]], axis=1))
    a_out[...] = jnp.dot(h, wa[...], preferred_element_type=jnp.float32) + b[...]
    u_out[...] = jnp.dot(h, wb[...], preferred_element_type=jnp.float32)


@functools.lru_cache(maxsize=None)
def _make_stage3(n_pad):
    npb = n_pad // _BLK
    return pl.pallas_call(
        _stage3_body,
        grid=(npb,),
        in_specs=[_rowblk(32), _rowblk(16), _rowblk_hi(16, npb),
                  _full((32, 16)), _full((32, 16)), _full((1, 16))],
        out_specs=[_rowblk(16), _rowblk(16)],
        out_shape=[jax.ShapeDtypeStruct((n_pad, 16), jnp.float32),
                   jax.ShapeDtypeStruct((n_pad, 16), jnp.float32)],
    )


def _gmp_body(a3, bid, sum_out, cnt_out, sacc, cacc):
    i = pl.program_id(0)
    n = pl.num_programs(0)
    oh = (bid[...] == lax.broadcasted_iota(jnp.int32, (_BLK, _NB), 1)
          ).astype(jnp.float32)
    dn = (((0,), (0,)), ((), ()))
    s = lax.dot_general(oh, a3[...], dn, preferred_element_type=jnp.float32)
    c = lax.dot_general(oh, jnp.ones((_BLK, 16), jnp.float32), dn,
                        preferred_element_type=jnp.float32)

    @pl.when(i == 0)
    def _():
        sacc[...] = s
        cacc[...] = c

    @pl.when(i > 0)
    def _():
        sacc[...] += s
        cacc[...] += c

    @pl.when(i == n - 1)
    def _():
        sum_out[...] = sacc[...]
        cnt_out[...] = cacc[...]


@functools.lru_cache(maxsize=None)
def _make_gmp(n_pad):
    return pl.pallas_call(
        _gmp_body,
        grid=(n_pad // _BLK,),
        in_specs=[_rowblk(16), pl.BlockSpec((_BLK, 1), lambda i: (i, 0))],
        out_specs=[pl.BlockSpec((_NB, 16), lambda i: (0, 0)),
                   pl.BlockSpec((_NB, 16), lambda i: (0, 0))],
        out_shape=[jax.ShapeDtypeStruct((_NB, 16), jnp.float32),
                   jax.ShapeDtypeStruct((_NB, 16), jnp.float32)],
        scratch_shapes=[pltpu.VMEM((_NB, 16), jnp.float32),
                        pltpu.VMEM((_NB, 16), jnp.float32)],
    )


def _epsum(e):
    acc = e[pl.ds(0, _NB), :]
    for t in range(1, 2 * _NSUB):
        acc = acc + e[pl.ds(t * _NB, _NB), :]
    return acc


def _final_body(s0, c0, e0, s1, c1, e1, s2, c2, e2, w0, w1, w2, b, out):
    m0 = (s0[...] + _epsum(e0)) / jnp.maximum(c0[...], 1.0)
    m1 = (s1[...] + _epsum(e1)) / jnp.maximum(c1[...], 1.0)
    m2 = (s2[...] + _epsum(e2)) / jnp.maximum(c2[...], 1.0)
    lg = (jnp.dot(m0, w0[...], preferred_element_type=jnp.float32)
          + jnp.dot(m1, w1[...], preferred_element_type=jnp.float32)
          + jnp.dot(m2, w2[...], preferred_element_type=jnp.float32)
          + b[...])
    col = lax.broadcasted_iota(jnp.int32, (_NB, 16), 1)
    lg = jnp.where(col < _OUT, lg, -1e30)
    mx = jnp.max(lg, axis=1, keepdims=True)
    e = jnp.exp(lg - mx)
    out[...] = e / jnp.sum(e, axis=1, keepdims=True)


@functools.lru_cache(maxsize=None)
def _make_final():
    return pl.pallas_call(
        _final_body,
        out_shape=jax.ShapeDtypeStruct((_NB, 16), jnp.float32),
    )


def _branch(x, idx, val, bid, w1, c1, w2, c2, w3, c3):
    n, d = x.shape
    nnz = val.shape[0]
    n_pad = _rup(n, _NSUB * _C)
    e_pad = _rup(nnz, _NSUB * _NCORE * _SUPE)

    xp = jnp.pad(x, ((0, n_pad - n), (0, 0)))
    rows = jnp.pad(idx[0], (0, e_pad - nnz)).astype(jnp.int32)
    rows2 = rows.reshape(-1, _C)
    cols = jnp.pad(idx[1], (0, e_pad - nnz)).astype(jnp.int32)
    vals = jnp.pad(val, (0, e_pad - nnz))
    bidp = jnp.pad(bid, (0, n_pad - n), constant_values=_NB)
    bidp = bidp.astype(jnp.int32).reshape(n_pad, 1)

    a1, u1lo, u1hi = _make_stage1(n_pad)(
        xp, w1[:d], w1[d:, :16], w1[d:, 16:], c1.reshape(1, 32))
    y1 = _make_spmm(n_pad, e_pad, False)(u1lo, u1hi, rows2, cols, vals)

    a2, u2lo, u2hi = _make_stage2(n_pad)(
        a1, y1, y1, w2[:_CONV], w2[_CONV:, :16], w2[_CONV:, 16:],
        c2.reshape(1, 32))
    y2 = _make_spmm(n_pad, e_pad, False)(u2lo, u2hi, rows2, cols, vals)

    w3a = jnp.pad(w3[:_CONV], ((0, 0), (0, 16 - _OUT)))
    w3b = jnp.pad(w3[_CONV:], ((0, 0), (0, 16 - _OUT)))
    b3p = jnp.pad(c3, (0, 16 - _OUT)).reshape(1, 16)
    a3, u3 = _make_stage3(n_pad)(a2, y2, y2, w3a, w3b, b3p)
    bt16 = jnp.tile(bidp, (1, 16))
    ep = _make_edgepool(n_pad, e_pad)(u3, bt16, rows, cols, vals)
    ep = ep.reshape(2 * _NSUB * _NB, 16)

    nsum, cnt = _make_gmp(n_pad)(a3, bidp)
    return nsum, cnt, ep


def kernel(X0, L0_indices, L0_values, batch0,
           X1, L1_indices, L1_values, batch1,
           X2, L2_indices, L2_values, batch2,
           W0_1, b0_1, W0_2, b0_2, W0_3, b0_3,
           W1_1, b1_1, W1_2, b1_2, W1_3, b1_3,
           W2_1, b2_1, W2_2, b2_2, W2_3, b2_3,
           Wf, bf):
    s0, c0, e0 = _branch(X0, L0_indices, L0_values, batch0,
                         W0_1, b0_1, W0_2, b0_2, W0_3, b0_3)
    s1, c1, e1 = _branch(X1, L1_indices, L1_values, batch1,
                         W1_1, b1_1, W1_2, b1_2, W1_3, b1_3)
    s2, c2, e2 = _branch(X2, L2_indices, L2_values, batch2,
                         W2_1, b2_1, W2_2, b2_2, W2_3, b2_3)

    wfp = [jnp.pad(Wf[10 * k:10 * (k + 1)], ((0, 6), (0, 6)))
           for k in range(3)]
    bfp = jnp.pad(bf, (0, 6)).reshape(1, 16)
    out = _make_final()(s0, c0, e0, s1, c1, e1, s2, c2, e2,
                        wfp[0], wfp[1], wfp[2], bfp)
    return out[:, :_OUT]


# speedup vs baseline: 1.1060x; 1.1046x over previous
import functools

import jax
import jax.numpy as jnp
from jax import lax
from jax.experimental import pallas as pl
from jax.experimental.pallas import tpu as pltpu
import jax.experimental.pallas.tpu_sc as plsc

_CONV, _OUT, _NB = 32, 10, 32
_BLK = 1024
_C = 128
_NSUB = 16
_NCORE = 2


def _rup(x, m):
    return (x + m - 1) // m * m


def _lrelu(t):
    return jnp.where(t > 0, t, 0.01 * t)


_SUP = 8
_SUPE = _SUP * _C
_HALF = _SUP // 2 * _C


@functools.lru_cache(maxsize=None)
def _make_spmm(n_pad, e_pad, esplit):
    eps = e_pad // (_NSUB * _NCORE) if esplit else e_pad // _NSUB
    nblocks = eps // _SUPE
    nz = n_pad // (_NSUB * _C)
    rows_per_sub = n_pad // _NSUB
    erow_total = e_pad // _C
    mesh = plsc.VectorSubcoreMesh(
        core_axis_name="c", subcore_axis_name="s",
        num_cores=_NCORE, num_subcores=_NSUB)

    @functools.partial(
        pl.kernel,
        out_type=jax.ShapeDtypeStruct((2 * n_pad, 16), jnp.float32),
        mesh=mesh,
        scratch_types=[
            pltpu.VMEM_SHARED((n_pad, 16), jnp.float32),
            pltpu.VMEM((_SUPE,), jnp.int32),
            pltpu.VMEM((_SUP, _C), jnp.int32),
            pltpu.VMEM((_SUPE,), jnp.float32),
            pltpu.VMEM((_C, 16), jnp.float32),
            pltpu.VMEM((_C, 16), jnp.float32),
            pltpu.VMEM((_C, 16), jnp.float32),
            pltpu.SemaphoreType.DMA,
            pltpu.SemaphoreType.DMA,
            pltpu.SemaphoreType.DMA,
        ],
        compiler_params=pltpu.CompilerParams(use_tc_tiling_on_sc=False),
    )
    def spmm(u0, u1, rows2, cols, vals, y, acc, colv, rowv8, valv, gbufa,
             gbufb, zbuf, semi, sema, semb):
        cid = lax.axis_index("c")
        sid = lax.axis_index("s")
        rbase = sid * rows_per_sub
        for r in range(_C):
            zbuf[r] = jnp.zeros((16,), jnp.float32)
        for j in range(nz):
            pltpu.sync_copy(zbuf, acc.at[pl.ds(rbase + j * _C, _C)])
        plsc.subcore_barrier()

        ebase = sid * eps + cid * (e_pad // 2 if esplit else 0)
        ebrow = sid * (eps // _C) + cid * (e_pad // 2 // _C if esplit else 0)
        gbufs = (gbufa, gbufb)
        gsems = (sema, semb)

        def stage(eoff, eroff, half):
            co = half * _HALF
            return [
                pltpu.async_copy(cols.at[pl.ds(eoff, _HALF)],
                                 colv.at[pl.ds(co, _HALF)], semi),
                pltpu.async_copy(vals.at[pl.ds(eoff, _HALF)],
                                 valv.at[pl.ds(co, _HALF)], semi),
                pltpu.async_copy(rows2.at[pl.ds(eroff, _SUP // 2)],
                                 rowv8.at[pl.ds(half * (_SUP // 2), _SUP // 2)],
                                 semi),
            ]

        def issue_gather(k, buf, sem):
            idxref = colv.at[pl.ds(k * _C, _C)]

            @pl.when(cid == 0)
            def _():
                pltpu.async_copy(u0.at[idxref], buf, sem)

            @pl.when(cid == 1)
            def _():
                pltpu.async_copy(u1.at[idxref], buf, sem)

            return pltpu.make_async_copy(u0.at[idxref], buf, sem)

        def process(k, buf):
            base = k * _C
            for g in range(_C // 16):
                vv = valv[pl.ds(base + g * 16, 16)]
                for l in range(16):
                    e = g * 16 + l
                    buf[e] = buf[e] * vv[l]
            pltpu.sync_copy(buf, acc.at[rowv8.at[k]], add=True)

        for dd in stage(ebase, ebrow, 0):
            dd.wait()
        issue_gather(0, gbufa, sema)

        def block(b, carry):
            eoff = ebase + b * _SUPE
            eroff = ebrow + b * _SUP
            hb_descs = stage(eoff + _HALF, eroff + _SUP // 2, 1)
            nxt = jnp.minimum(eoff + _SUPE, e_pad - _HALF)
            nxtr = jnp.minimum(eroff + _SUP, erow_total - _SUP // 2)
            na_descs = None
            d = [pltpu.make_async_copy(u0.at[colv.at[pl.ds(0, _C)]],
                                       gbufa, sema)] + [None] * (_SUP - 1)
            for k in range(_SUP):
                buf = gbufs[k % 2]
                if k == _SUP // 2 - 1:
                    for dd in hb_descs:
                        dd.wait()
                if k < _SUP - 1:
                    d[k + 1] = issue_gather(k + 1, gbufs[(k + 1) % 2],
                                            gsems[(k + 1) % 2])
                if k == _SUP // 2:
                    na_descs = stage(nxt, nxtr, 0)
                if k == _SUP - 1:
                    for dd in na_descs:
                        dd.wait()
                    issue_gather(0, gbufa, sema)
                d[k].wait()
                process(k, buf)
            return carry

        lax.fori_loop(0, nblocks, block, 0)
        pltpu.make_async_copy(u0.at[colv.at[pl.ds(0, _C)]], gbufa, sema).wait()

        plsc.subcore_barrier()
        ybase = cid * n_pad + rbase
        for j in range(nz):
            pltpu.sync_copy(acc.at[pl.ds(rbase + j * _C, _C)],
                            y.at[pl.ds(ybase + j * _C, _C)])

    return spmm


def _full(shape):
    return pl.BlockSpec(shape, lambda i: (0, 0))


def _rowblk(w):
    return pl.BlockSpec((_BLK, w), lambda i: (i, 0))


def _rowblk_hi(w, npb):
    return pl.BlockSpec((_BLK, w), lambda i: (i + npb, 0))


def _stage1_body(x, wa, wblo, wbhi, b, a_out, ulo, uhi):
    xb = x[...]
    a_out[...] = jnp.dot(xb, wa[...], preferred_element_type=jnp.float32) + b[...]
    ulo[...] = jnp.dot(xb, wblo[...], preferred_element_type=jnp.float32)
    uhi[...] = jnp.dot(xb, wbhi[...], preferred_element_type=jnp.float32)


@functools.lru_cache(maxsize=None)
def _make_stage1(n_pad):
    return pl.pallas_call(
        _stage1_body,
        grid=(n_pad // _BLK,),
        in_specs=[_rowblk(64), _full((64, 32)), _full((64, 16)),
                  _full((64, 16)), _full((1, 32))],
        out_specs=[_rowblk(32), _rowblk(16), _rowblk(16)],
        out_shape=[jax.ShapeDtypeStruct((n_pad, 32), jnp.float32),
                   jax.ShapeDtypeStruct((n_pad, 16), jnp.float32),
                   jax.ShapeDtypeStruct((n_pad, 16), jnp.float32)],
    )


def _stage2_body(a_in, ylo, yhi, wa, wblo, wbhi, b, a_out, ulo, uhi):
    h = _lrelu(a_in[...] + jnp.concatenate([ylo[...], yhi[...]], axis=1))
    a_out[...] = jnp.dot(h, wa[...], preferred_element_type=jnp.float32) + b[...]
    ulo[...] = jnp.dot(h, wblo[...], preferred_element_type=jnp.float32)
    uhi[...] = jnp.dot(h, wbhi[...], preferred_element_type=jnp.float32)


@functools.lru_cache(maxsize=None)
def _make_stage2(n_pad):
    npb = n_pad // _BLK
    return pl.pallas_call(
        _stage2_body,
        grid=(npb,),
        in_specs=[_rowblk(32), _rowblk(16), _rowblk_hi(16, npb),
                  _full((32, 32)), _full((32, 16)), _full((32, 16)),
                  _full((1, 32))],
        out_specs=[_rowblk(32), _rowblk(16), _rowblk(16)],
        out_shape=[jax.ShapeDtypeStruct((n_pad, 32), jnp.float32),
                   jax.ShapeDtypeStruct((n_pad, 16), jnp.float32),
                   jax.ShapeDtypeStruct((n_pad, 16), jnp.float32)],
    )


def _stage3_body(a_in, ylo, yhi, wa, wb, b, a_out, u_out):
    h = _lrelu(a_in[...] + jnp.concatenate([ylo[...], yhi[...]], axis=1))
    a_out[...] = jnp.dot(h, wa[...], preferred_element_type=jnp.float32) + b[...]
    u_out[...] = jnp.dot(h, wb[...], preferred_element_type=jnp.float32)


@functools.lru_cache(maxsize=None)
def _make_stage3(n_pad):
    npb = n_pad // _BLK
    return pl.pallas_call(
        _stage3_body,
        grid=(npb,),
        in_specs=[_rowblk(32), _rowblk(16), _rowblk_hi(16, npb),
                  _full((32, 16)), _full((32, 16)), _full((1, 16))],
        out_specs=[_rowblk(16), _rowblk(16)],
        out_shape=[jax.ShapeDtypeStruct((n_pad, 16), jnp.float32),
                   jax.ShapeDtypeStruct((n_pad, 16), jnp.float32)],
    )


def _gmp_body(a3, y0, y1, bid, out, sacc, cacc):
    i = pl.program_id(0)
    n = pl.num_programs(0)
    h3 = a3[...] + y0[...] + y1[...]
    oh = (bid[...] == lax.broadcasted_iota(jnp.int32, (_BLK, _NB), 1)
          ).astype(jnp.float32)
    dn = (((0,), (0,)), ((), ()))
    s = lax.dot_general(oh, h3, dn, preferred_element_type=jnp.float32)
    c = lax.dot_general(oh, jnp.ones((_BLK, 16), jnp.float32), dn,
                        preferred_element_type=jnp.float32)

    @pl.when(i == 0)
    def _():
        sacc[...] = s
        cacc[...] = c

    @pl.when(i > 0)
    def _():
        sacc[...] += s
        cacc[...] += c

    @pl.when(i == n - 1)
    def _():
        out[...] = sacc[...] / jnp.maximum(cacc[...], 1.0)


@functools.lru_cache(maxsize=None)
def _make_gmp(n_pad):
    npb = n_pad // _BLK
    return pl.pallas_call(
        _gmp_body,
        grid=(npb,),
        in_specs=[_rowblk(16), _rowblk(16), _rowblk_hi(16, npb),
                  pl.BlockSpec((_BLK, 1), lambda i: (i, 0))],
        out_specs=pl.BlockSpec((_NB, 16), lambda i: (0, 0)),
        out_shape=jax.ShapeDtypeStruct((_NB, 16), jnp.float32),
        scratch_shapes=[pltpu.VMEM((_NB, 16), jnp.float32),
                        pltpu.VMEM((_NB, 16), jnp.float32)],
    )


def _final_body(m0, m1, m2, w0, w1, w2, b, out):
    lg = (jnp.dot(m0[...], w0[...], preferred_element_type=jnp.float32)
          + jnp.dot(m1[...], w1[...], preferred_element_type=jnp.float32)
          + jnp.dot(m2[...], w2[...], preferred_element_type=jnp.float32)
          + b[...])
    col = lax.broadcasted_iota(jnp.int32, (_NB, 16), 1)
    lg = jnp.where(col < _OUT, lg, -1e30)
    mx = jnp.max(lg, axis=1, keepdims=True)
    e = jnp.exp(lg - mx)
    out[...] = e / jnp.sum(e, axis=1, keepdims=True)


@functools.lru_cache(maxsize=None)
def _make_final():
    return pl.pallas_call(
        _final_body,
        out_shape=jax.ShapeDtypeStruct((_NB, 16), jnp.float32),
    )


def _branch(x, idx, val, bid, w1, c1, w2, c2, w3, c3):
    n, d = x.shape
    nnz = val.shape[0]
    n_pad = _rup(n, _NSUB * _C)
    e_pad = _rup(nnz, _NSUB * _NCORE * _SUPE)

    xp = jnp.pad(x, ((0, n_pad - n), (0, 0)))
    rows = jnp.pad(idx[0], (0, e_pad - nnz)).astype(jnp.int32)
    rows2 = rows.reshape(-1, _C)
    cols = jnp.pad(idx[1], (0, e_pad - nnz)).astype(jnp.int32)
    vals = jnp.pad(val, (0, e_pad - nnz))
    bidp = jnp.pad(bid, (0, n_pad - n), constant_values=_NB)
    bidp = bidp.astype(jnp.int32).reshape(n_pad, 1)

    a1, u1lo, u1hi = _make_stage1(n_pad)(
        xp, w1[:d], w1[d:, :16], w1[d:, 16:], c1.reshape(1, 32))
    y1 = _make_spmm(n_pad, e_pad, False)(u1lo, u1hi, rows2, cols, vals)

    a2, u2lo, u2hi = _make_stage2(n_pad)(
        a1, y1, y1, w2[:_CONV], w2[_CONV:, :16], w2[_CONV:, 16:],
        c2.reshape(1, 32))
    y2 = _make_spmm(n_pad, e_pad, False)(u2lo, u2hi, rows2, cols, vals)

    w3a = jnp.pad(w3[:_CONV], ((0, 0), (0, 16 - _OUT)))
    w3b = jnp.pad(w3[_CONV:], ((0, 0), (0, 16 - _OUT)))
    b3p = jnp.pad(c3, (0, 16 - _OUT)).reshape(1, 16)
    a3, u3 = _make_stage3(n_pad)(a2, y2, y2, w3a, w3b, b3p)
    y3 = _make_spmm(n_pad, e_pad, True)(u3, u3, rows2, cols, vals)

    return _make_gmp(n_pad)(a3, y3, y3, bidp)


def kernel(X0, L0_indices, L0_values, batch0,
           X1, L1_indices, L1_values, batch1,
           X2, L2_indices, L2_values, batch2,
           W0_1, b0_1, W0_2, b0_2, W0_3, b0_3,
           W1_1, b1_1, W1_2, b1_2, W1_3, b1_3,
           W2_1, b2_1, W2_2, b2_2, W2_3, b2_3,
           Wf, bf):
    m0 = _branch(X0, L0_indices, L0_values, batch0,
                 W0_1, b0_1, W0_2, b0_2, W0_3, b0_3)
    m1 = _branch(X1, L1_indices, L1_values, batch1,
                 W1_1, b1_1, W1_2, b1_2, W1_3, b1_3)
    m2 = _branch(X2, L2_indices, L2_values, batch2,
                 W2_1, b2_1, W2_2, b2_2, W2_3, b2_3)

    wfp = [jnp.pad(Wf[10 * k:10 * (k + 1)], ((0, 6), (0, 6)))
           for k in range(3)]
    bfp = jnp.pad(bf, (0, 6)).reshape(1, 16)
    out = _make_final()(m0, m1, m2, wfp[0], wfp[1], wfp[2], bfp)
    return out[:, :_OUT]


# speedup vs baseline: 1.3379x; 1.2097x over previous
import functools

import jax
import jax.numpy as jnp
from jax import lax
from jax.experimental import pallas as pl
from jax.experimental.pallas import tpu as pltpu
import jax.experimental.pallas.tpu_sc as plsc

_CONV, _OUT, _NB = 32, 10, 32
_BLK = 1024
_C = 128
_NSUB = 16
_NCORE = 2


def _rup(x, m):
    return (x + m - 1) // m * m


def _lrelu(t):
    return jnp.where(t > 0, t, 0.01 * t)


_SUP = 8
_SUPE = _SUP * _C
_HALF = _SUP // 2 * _C


@functools.lru_cache(maxsize=None)
def _make_spmm(n_pad, e_pad, esplit):
    eps = e_pad // (_NSUB * _NCORE) if esplit else e_pad // _NSUB
    nblocks = eps // _SUPE
    nz = n_pad // (_NSUB * _C)
    rows_per_sub = n_pad // _NSUB
    erow_total = e_pad // _C
    mesh = plsc.VectorSubcoreMesh(
        core_axis_name="c", subcore_axis_name="s",
        num_cores=_NCORE, num_subcores=_NSUB)

    @functools.partial(
        pl.kernel,
        out_type=jax.ShapeDtypeStruct((2 * n_pad, 16), jnp.float32),
        mesh=mesh,
        scratch_types=[
            pltpu.VMEM_SHARED((n_pad, 16), jnp.float32),
            pltpu.VMEM((_SUPE,), jnp.int32),
            pltpu.VMEM((_SUP, _C), jnp.int32),
            pltpu.VMEM((_SUPE,), jnp.float32),
            pltpu.VMEM((_C, 16), jnp.float32),
            pltpu.VMEM((_C, 16), jnp.float32),
            pltpu.VMEM((_C, 16), jnp.float32),
            pltpu.SemaphoreType.DMA,
            pltpu.SemaphoreType.DMA,
            pltpu.SemaphoreType.DMA,
        ],
        compiler_params=pltpu.CompilerParams(use_tc_tiling_on_sc=False),
    )
    def spmm(u0, u1, rows2, cols, vals, y, acc, colv, rowv8, valv, gbufa,
             gbufb, zbuf, semi, sema, semb):
        cid = lax.axis_index("c")
        sid = lax.axis_index("s")
        rbase = sid * rows_per_sub
        for r in range(_C):
            zbuf[r] = jnp.zeros((16,), jnp.float32)
        for j in range(nz):
            pltpu.sync_copy(zbuf, acc.at[pl.ds(rbase + j * _C, _C)])
        plsc.subcore_barrier()

        ebase = sid * eps + cid * (e_pad // 2 if esplit else 0)
        ebrow = sid * (eps // _C) + cid * (e_pad // 2 // _C if esplit else 0)
        gbufs = (gbufa, gbufb)
        gsems = (sema, semb)

        def stage(eoff, eroff, half):
            co = half * _HALF
            return [
                pltpu.async_copy(cols.at[pl.ds(eoff, _HALF)],
                                 colv.at[pl.ds(co, _HALF)], semi),
                pltpu.async_copy(vals.at[pl.ds(eoff, _HALF)],
                                 valv.at[pl.ds(co, _HALF)], semi),
                pltpu.async_copy(rows2.at[pl.ds(eroff, _SUP // 2)],
                                 rowv8.at[pl.ds(half * (_SUP // 2), _SUP // 2)],
                                 semi),
            ]

        def issue_gather(k, buf, sem):
            idxref = colv.at[pl.ds(k * _C, _C)]

            @pl.when(cid == 0)
            def _():
                pltpu.async_copy(u0.at[idxref], buf, sem)

            @pl.when(cid == 1)
            def _():
                pltpu.async_copy(u1.at[idxref], buf, sem)

            return pltpu.make_async_copy(u0.at[idxref], buf, sem)

        def process(k, buf):
            base = k * _C
            for g in range(_C // 16):
                vv = valv[pl.ds(base + g * 16, 16)]
                for l in range(16):
                    e = g * 16 + l
                    buf[e] = buf[e] * vv[l]
            pltpu.sync_copy(buf, acc.at[rowv8.at[k]], add=True)

        for dd in stage(ebase, ebrow, 0):
            dd.wait()
        issue_gather(0, gbufa, sema)

        def block(b, carry):
            eoff = ebase + b * _SUPE
            eroff = ebrow + b * _SUP
            hb_descs = stage(eoff + _HALF, eroff + _SUP // 2, 1)
            nxt = jnp.minimum(eoff + _SUPE, e_pad - _HALF)
            nxtr = jnp.minimum(eroff + _SUP, erow_total - _SUP // 2)
            na_descs = None
            d = [pltpu.make_async_copy(u0.at[colv.at[pl.ds(0, _C)]],
                                       gbufa, sema)] + [None] * (_SUP - 1)
            for k in range(_SUP):
                buf = gbufs[k % 2]
                if k == _SUP // 2 - 1:
                    for dd in hb_descs:
                        dd.wait()
                if k < _SUP - 1:
                    d[k + 1] = issue_gather(k + 1, gbufs[(k + 1) % 2],
                                            gsems[(k + 1) % 2])
                if k == _SUP // 2:
                    na_descs = stage(nxt, nxtr, 0)
                if k == _SUP - 1:
                    for dd in na_descs:
                        dd.wait()
                    issue_gather(0, gbufa, sema)
                d[k].wait()
                process(k, buf)
            return carry

        lax.fori_loop(0, nblocks, block, 0)
        pltpu.make_async_copy(u0.at[colv.at[pl.ds(0, _C)]], gbufa, sema).wait()

        plsc.subcore_barrier()
        ybase = cid * n_pad + rbase
        for j in range(nz):
            pltpu.sync_copy(acc.at[pl.ds(rbase + j * _C, _C)],
                            y.at[pl.ds(ybase + j * _C, _C)])

    return spmm


def _full(shape):
    return pl.BlockSpec(shape, lambda i: (0, 0))


def _rowblk(w):
    return pl.BlockSpec((_BLK, w), lambda i: (i, 0))


def _rowblk_hi(w, npb):
    return pl.BlockSpec((_BLK, w), lambda i: (i + npb, 0))


def _stage1_body(x, wa, wblo, wbhi, b, a_out, ulo, uhi):
    xb = x[...]
    a_out[...] = jnp.dot(xb, wa[...], preferred_element_type=jnp.float32) + b[...]
    ulo[...] = jnp.dot(xb, wblo[...], preferred_element_type=jnp.float32)
    uhi[...] = jnp.dot(xb, wbhi[...], preferred_element_type=jnp.float32)


@functools.lru_cache(maxsize=None)
def _make_stage1(n_pad):
    return pl.pallas_call(
        _stage1_body,
        grid=(n_pad // _BLK,),
        in_specs=[_rowblk(64), _full((64, 32)), _full((64, 16)),
                  _full((64, 16)), _full((1, 32))],
        out_specs=[_rowblk(32), _rowblk(16), _rowblk(16)],
        out_shape=[jax.ShapeDtypeStruct((n_pad, 32), jnp.float32),
                   jax.ShapeDtypeStruct((n_pad, 16), jnp.float32),
                   jax.ShapeDtypeStruct((n_pad, 16), jnp.float32)],
    )


def _stage2_body(a_in, ylo, yhi, wa, wblo, wbhi, b, a_out, ulo, uhi):
    h = _lrelu(a_in[...] + jnp.concatenate([ylo[...], yhi[...]], axis=1))
    a_out[...] = jnp.dot(h, wa[...], preferred_element_type=jnp.float32) + b[...]
    ulo[...] = jnp.dot(h, wblo[...], preferred_element_type=jnp.float32)
    uhi[...] = jnp.dot(h, wbhi[...], preferred_element_type=jnp.float32)


@functools.lru_cache(maxsize=None)
def _make_stage2(n_pad):
    npb = n_pad // _BLK
    return pl.pallas_call(
        _stage2_body,
        grid=(npb,),
        in_specs=[_rowblk(32), _rowblk(16), _rowblk_hi(16, npb),
                  _full((32, 32)), _full((32, 16)), _full((32, 16)),
                  _full((1, 32))],
        out_specs=[_rowblk(32), _rowblk(16), _rowblk(16)],
        out_shape=[jax.ShapeDtypeStruct((n_pad, 32), jnp.float32),
                   jax.ShapeDtypeStruct((n_pad, 16), jnp.float32),
                   jax.ShapeDtypeStruct((n_pad, 16), jnp.float32)],
    )


def _stage3_body(a_in, ylo, yhi, wa, wb, b, a_out, u_out):
    h = _lrelu(a_in[...] + jnp.concatenate([ylo[...], yhi[...]], axis=1))
    a_out[...] = jnp.dot(h, wa[...], preferred_element_type=jnp.float32) + b[...]
    u_out[...] = jnp.dot(h, wb[...], preferred_element_type=jnp.float32)


@functools.lru_cache(maxsize=None)
def _make_stage3(n_pad):
    npb = n_pad // _BLK
    return pl.pallas_call(
        _stage3_body,
        grid=(npb,),
        in_specs=[_rowblk(32), _rowblk(16), _rowblk_hi(16, npb),
                  _full((32, 16)), _full((32, 16)), _full((1, 16))],
        out_specs=[_rowblk(16), _rowblk(16)],
        out_shape=[jax.ShapeDtypeStruct((n_pad, 16), jnp.float32),
                   jax.ShapeDtypeStruct((n_pad, 16), jnp.float32)],
    )


def _gmp_body(a3, y0, y1, bid, out, sacc, cacc):
    i = pl.program_id(0)
    n = pl.num_programs(0)
    h3 = a3[...] + y0[...] + y1[...]
    oh = (bid[...] == lax.broadcasted_iota(jnp.int32, (_BLK, _NB), 1)
          ).astype(jnp.float32)
    dn = (((0,), (0,)), ((), ()))
    s = lax.dot_general(oh, h3, dn, preferred_element_type=jnp.float32)
    c = lax.dot_general(oh, jnp.ones((_BLK, 16), jnp.float32), dn,
                        preferred_element_type=jnp.float32)

    @pl.when(i == 0)
    def _():
        sacc[...] = s
        cacc[...] = c

    @pl.when(i > 0)
    def _():
        sacc[...] += s
        cacc[...] += c

    @pl.when(i == n - 1)
    def _():
        out[...] = sacc[...] / jnp.maximum(cacc[...], 1.0)


@functools.lru_cache(maxsize=None)
def _make_gmp(n_pad):
    npb = n_pad // _BLK
    return pl.pallas_call(
        _gmp_body,
        grid=(npb,),
        in_specs=[_rowblk(16), _rowblk(16), _rowblk_hi(16, npb),
                  pl.BlockSpec((_BLK, 1), lambda i: (i, 0))],
        out_specs=pl.BlockSpec((_NB, 16), lambda i: (0, 0)),
        out_shape=jax.ShapeDtypeStruct((_NB, 16), jnp.float32),
        scratch_shapes=[pltpu.VMEM((_NB, 16), jnp.float32),
                        pltpu.VMEM((_NB, 16), jnp.float32)],
    )


def _final_body(m0, m1, m2, w0, w1, w2, b, out):
    lg = (jnp.dot(m0[...], w0[...], preferred_element_type=jnp.float32)
          + jnp.dot(m1[...], w1[...], preferred_element_type=jnp.float32)
          + jnp.dot(m2[...], w2[...], preferred_element_type=jnp.float32)
          + b[...])
    col = lax.broadcasted_iota(jnp.int32, (_NB, 16), 1)
    lg = jnp.where(col < _OUT, lg, -1e30)
    mx = jnp.max(lg, axis=1, keepdims=True)
    e = jnp.exp(lg - mx)
    out[...] = e / jnp.sum(e, axis=1, keepdims=True)


@functools.lru_cache(maxsize=None)
def _make_final():
    return pl.pallas_call(
        _final_body,
        out_shape=jax.ShapeDtypeStruct((_NB, 16), jnp.float32),
    )


def _branch(x, idx, val, bid, w1, c1, w2, c2, w3, c3):
    n, d = x.shape
    nnz = val.shape[0]
    n_pad = _rup(n, _NSUB * _C)
    e_pad = _rup(nnz, _NSUB * _SUPE)
    e_pad_e = _rup(nnz, _NSUB * _NCORE * _SUPE)

    xp = jnp.pad(x, ((0, n_pad - n), (0, 0)))
    rows2 = jnp.pad(idx[0], (0, e_pad - nnz)).astype(jnp.int32).reshape(-1, _C)
    cols = jnp.pad(idx[1], (0, e_pad - nnz)).astype(jnp.int32)
    vals = jnp.pad(val, (0, e_pad - nnz))
    rows2e = jnp.pad(idx[0], (0, e_pad_e - nnz)).astype(jnp.int32)
    rows2e = rows2e.reshape(-1, _C)
    colse = jnp.pad(idx[1], (0, e_pad_e - nnz)).astype(jnp.int32)
    valse = jnp.pad(val, (0, e_pad_e - nnz))
    bidp = jnp.pad(bid, (0, n_pad - n), constant_values=_NB)
    bidp = bidp.astype(jnp.int32).reshape(n_pad, 1)

    a1, u1lo, u1hi = _make_stage1(n_pad)(
        xp, w1[:d], w1[d:, :16], w1[d:, 16:], c1.reshape(1, 32))
    y1 = _make_spmm(n_pad, e_pad, False)(u1lo, u1hi, rows2, cols, vals)

    a2, u2lo, u2hi = _make_stage2(n_pad)(
        a1, y1, y1, w2[:_CONV], w2[_CONV:, :16], w2[_CONV:, 16:],
        c2.reshape(1, 32))
    y2 = _make_spmm(n_pad, e_pad, False)(u2lo, u2hi, rows2, cols, vals)

    w3a = jnp.pad(w3[:_CONV], ((0, 0), (0, 16 - _OUT)))
    w3b = jnp.pad(w3[_CONV:], ((0, 0), (0, 16 - _OUT)))
    b3p = jnp.pad(c3, (0, 16 - _OUT)).reshape(1, 16)
    a3, u3 = _make_stage3(n_pad)(a2, y2, y2, w3a, w3b, b3p)
    y3 = _make_spmm(n_pad, e_pad_e, True)(u3, u3, rows2e, colse, valse)

    return _make_gmp(n_pad)(a3, y3, y3, bidp)


def kernel(X0, L0_indices, L0_values, batch0,
           X1, L1_indices, L1_values, batch1,
           X2, L2_indices, L2_values, batch2,
           W0_1, b0_1, W0_2, b0_2, W0_3, b0_3,
           W1_1, b1_1, W1_2, b1_2, W1_3, b1_3,
           W2_1, b2_1, W2_2, b2_2, W2_3, b2_3,
           Wf, bf):
    m0 = _branch(X0, L0_indices, L0_values, batch0,
                 W0_1, b0_1, W0_2, b0_2, W0_3, b0_3)
    m1 = _branch(X1, L1_indices, L1_values, batch1,
                 W1_1, b1_1, W1_2, b1_2, W1_3, b1_3)
    m2 = _branch(X2, L2_indices, L2_values, batch2,
                 W2_1, b2_1, W2_2, b2_2, W2_3, b2_3)

    wfp = [jnp.pad(Wf[10 * k:10 * (k + 1)], ((0, 6), (0, 6)))
           for k in range(3)]
    bfp = jnp.pad(bf, (0, 6)).reshape(1, 16)
    out = _make_final()(m0, m1, m2, wfp[0], wfp[1], wfp[2], bfp)
    return out[:, :_OUT]
